# baseline (speedup 1.0000x reference)
import functools

import jax
import jax.numpy as jnp
from jax import lax
from jax.experimental import pallas as pl
from jax.experimental.pallas import tpu as pltpu
from jax.experimental.pallas import tpu_sc as plsc

EMB = 32
NCLS = 100
B = 4096
L = 200
F1 = 128
F2 = L - F1
NC, NS = 2, 16
NW = NC * NS
BPW = B // NW

_mesh = plsc.VectorSubcoreMesh(core_axis_name="c", subcore_axis_name="s")


@functools.partial(
    pl.kernel,
    mesh=_mesh,
    out_type=jax.ShapeDtypeStruct((B, EMB), jnp.float32),
    compiler_params=pltpu.CompilerParams(use_tc_tiling_on_sc=False),
    scratch_types=[
        pltpu.VMEM((BPW, F1), jnp.int32),
        pltpu.VMEM((BPW, F2), jnp.int32),
        pltpu.VMEM((F1, EMB), jnp.float32),
        pltpu.VMEM((F2, EMB), jnp.float32),
        pltpu.VMEM((F1, EMB), jnp.float32),
        pltpu.VMEM((F2, EMB), jnp.float32),
        pltpu.VMEM((F1, EMB), jnp.float32),
        pltpu.VMEM((F2, EMB), jnp.float32),
        pltpu.VMEM((BPW, EMB), jnp.float32),
        pltpu.SemaphoreType.DMA,
        pltpu.SemaphoreType.DMA,
        pltpu.SemaphoreType.DMA,
    ],
)
def _pool_kernel(x_hbm, table_hbm, avg_hbm, idx_a, idx_b, b0a, b0b, b1a, b1b,
                 b2a, b2b, out_v, sem0, sem1, sem2):
    wid = lax.axis_index("s") * NC + lax.axis_index("c")
    row0 = wid * BPW
    pltpu.sync_copy(x_hbm.at[pl.ds(row0, BPW), pl.ds(0, F1)], idx_a)
    pltpu.sync_copy(x_hbm.at[pl.ds(row0, BPW), pl.ds(F1, F2)], idx_b)

    NB = 3
    bufs = ((b0a, b0b), (b1a, b1b), (b2a, b2b))
    sems = (sem0, sem1, sem2)
    inv_l = 1.0 / L

    for p in range(NB):
        pltpu.async_copy(table_hbm.at[idx_a.at[p]], bufs[p][0], sems[p])
        pltpu.async_copy(table_hbm.at[idx_b.at[p]], bufs[p][1], sems[p])

    def body(g, carry):
        for p in range(NB):
            b = NB * g + p

            @pl.when(b < BPW)
            def _():
                ba, bb = bufs[p]
                pltpu.make_async_copy(table_hbm.at[idx_a.at[0]], ba,
                                      sems[p]).wait()
                pltpu.make_async_copy(table_hbm.at[idx_b.at[0]], bb,
                                      sems[p]).wait()
                lo = [jnp.zeros((16,), jnp.float32) for _ in range(4)]
                hi = [jnp.zeros((16,), jnp.float32) for _ in range(4)]
                for buf, n in ((ba, F1), (bb, F2)):
                    for j in range(n):
                        c = j % 4
                        lo[c] = lo[c] + buf[j, pl.ds(0, 16)]
                        hi[c] = hi[c] + buf[j, pl.ds(16, 16)]
                out_v[b, pl.ds(0, 16)] = (
                    (lo[0] + lo[1]) + (lo[2] + lo[3])) * inv_l
                out_v[b, pl.ds(16, 16)] = (
                    (hi[0] + hi[1]) + (hi[2] + hi[3])) * inv_l

                @pl.when(b + NB < BPW)
                def _():
                    pltpu.async_copy(table_hbm.at[idx_a.at[b + NB]], ba,
                                     sems[p])
                    pltpu.async_copy(table_hbm.at[idx_b.at[b + NB]], bb,
                                     sems[p])
        return carry

    lax.fori_loop(0, (BPW + NB - 1) // NB, body, 0)
    pltpu.sync_copy(out_v, avg_hbm.at[pl.ds(wid * BPW, BPW)])


def _linear_body(avg_ref, wt_ref, bias_ref, out_ref):
    out_ref[...] = (
        jnp.dot(avg_ref[...], wt_ref[...], preferred_element_type=jnp.float32)
        + bias_ref[...]
    )


def kernel(x, table, W_out, b_out):
    avg = _pool_kernel(x.astype(jnp.int32), table)
    out = pl.pallas_call(
        _linear_body,
        out_shape=jax.ShapeDtypeStruct((B, NCLS), jnp.float32),
    )(avg, W_out.T, b_out.reshape(1, NCLS))
    return out


# speedup vs baseline: 1.0915x; 1.0915x over previous
import functools

import jax
import jax.numpy as jnp
from jax import lax
from jax.experimental import pallas as pl
from jax.experimental.pallas import tpu as pltpu
from jax.experimental.pallas import tpu_sc as plsc

EMB = 32
NCLS = 100
B = 4096
L = 200
F1 = 128
F2 = L - F1
NC, NS = 2, 16
NW = NC * NS
BPW = B // NW

_mesh = plsc.VectorSubcoreMesh(core_axis_name="c", subcore_axis_name="s")


@functools.partial(
    pl.kernel,
    mesh=_mesh,
    out_type=jax.ShapeDtypeStruct((B, EMB), jnp.float32),
    compiler_params=pltpu.CompilerParams(use_tc_tiling_on_sc=False),
    scratch_types=[
        pltpu.VMEM((BPW, F1), jnp.int32),
        pltpu.VMEM((BPW, F2), jnp.int32),
        pltpu.VMEM((F1, EMB), jnp.float32),
        pltpu.VMEM((F2, EMB), jnp.float32),
        pltpu.VMEM((F1, EMB), jnp.float32),
        pltpu.VMEM((F2, EMB), jnp.float32),
        pltpu.VMEM((BPW, EMB), jnp.float32),
        pltpu.SemaphoreType.DMA,
        pltpu.SemaphoreType.DMA,
    ],
)
def _pool_kernel(x_hbm, table_hbm, avg_hbm, idx_a, idx_b, b0a, b0b, b1a, b1b,
                 out_v, sem0, sem1):
    wid = lax.axis_index("s") * NC + lax.axis_index("c")
    row0 = wid * BPW
    pltpu.sync_copy(x_hbm.at[pl.ds(row0, BPW), pl.ds(0, F1)], idx_a)
    pltpu.sync_copy(x_hbm.at[pl.ds(row0, BPW), pl.ds(F1, F2)], idx_b)

    bufs = ((b0a, b0b), (b1a, b1b))
    sems = (sem0, sem1)
    inv_l = 1.0 / L

    for p in range(2):
        pltpu.async_copy(table_hbm.at[idx_a.at[p]], bufs[p][0], sems[p])
        pltpu.async_copy(table_hbm.at[idx_b.at[p]], bufs[p][1], sems[p])

    def body(g, carry):
        for p in range(2):
            b = 2 * g + p
            ba, bb = bufs[p]
            pltpu.make_async_copy(table_hbm.at[idx_a.at[0]], ba, sems[p]).wait()
            pltpu.make_async_copy(table_hbm.at[idx_b.at[0]], bb, sems[p]).wait()
            lo = [jnp.zeros((16,), jnp.float32) for _ in range(4)]
            hi = [jnp.zeros((16,), jnp.float32) for _ in range(4)]
            for buf, n in ((ba, F1), (bb, F2)):
                for j in range(n):
                    c = j % 4
                    lo[c] = lo[c] + buf[j, pl.ds(0, 16)]
                    hi[c] = hi[c] + buf[j, pl.ds(16, 16)]
            out_v[b, pl.ds(0, 16)] = ((lo[0] + lo[1]) + (lo[2] + lo[3])) * inv_l
            out_v[b, pl.ds(16, 16)] = ((hi[0] + hi[1]) + (hi[2] + hi[3])) * inv_l

            @pl.when(b + 2 < BPW)
            def _():
                pltpu.async_copy(table_hbm.at[idx_a.at[b + 2]], ba, sems[p])
                pltpu.async_copy(table_hbm.at[idx_b.at[b + 2]], bb, sems[p])
        return carry

    lax.fori_loop(0, BPW // 2, body, 0)
    pltpu.sync_copy(out_v, avg_hbm.at[pl.ds(wid * BPW, BPW)])


def _linear_body(avg_ref, wt_ref, bias_ref, out_ref):
    out_ref[...] = (
        jnp.dot(avg_ref[...], wt_ref[...], preferred_element_type=jnp.float32)
        + bias_ref[...]
    )


def kernel(x, table, W_out, b_out):
    avg = _pool_kernel(x.astype(jnp.int32), table)
    out = pl.pallas_call(
        _linear_body,
        out_shape=jax.ShapeDtypeStruct((B, NCLS), jnp.float32),
    )(avg, W_out.T, b_out.reshape(1, NCLS))
    return out


# speedup vs baseline: 1.0979x; 1.0059x over previous
import functools

import jax
import jax.numpy as jnp
from jax import lax
from jax.experimental import pallas as pl
from jax.experimental.pallas import tpu as pltpu
from jax.experimental.pallas import tpu_sc as plsc

EMB = 32
NCLS = 100
B = 4096
L = 200
F1 = 128
F2 = L - F1
NC, NS = 2, 16
NW = NC * NS
BPW = B // NW

_mesh = plsc.VectorSubcoreMesh(core_axis_name="c", subcore_axis_name="s")


@functools.partial(
    pl.kernel,
    mesh=_mesh,
    out_type=jax.ShapeDtypeStruct((B, EMB), jnp.float32),
    compiler_params=pltpu.CompilerParams(use_tc_tiling_on_sc=False),
    scratch_types=[
        pltpu.VMEM((BPW, F1), jnp.int32),
        pltpu.VMEM((BPW, F2), jnp.int32),
        pltpu.VMEM((F1, EMB), jnp.float32),
        pltpu.VMEM((F2, EMB), jnp.float32),
        pltpu.VMEM((F1, EMB), jnp.float32),
        pltpu.VMEM((F2, EMB), jnp.float32),
        pltpu.VMEM((BPW, EMB), jnp.float32),
        pltpu.SemaphoreType.DMA,
        pltpu.SemaphoreType.DMA,
        pltpu.SemaphoreType.DMA,
        pltpu.SemaphoreType.DMA,
    ],
)
def _pool_kernel(x_hbm, table_hbm, avg_hbm, idx_a, idx_b, b0a, b0b, b1a, b1b,
                 out_v, sa0, sa1, sb0, sb1):
    wid = lax.axis_index("s") * NC + lax.axis_index("c")
    row0 = wid * BPW
    pltpu.sync_copy(x_hbm.at[pl.ds(row0, BPW), pl.ds(0, F1)], idx_a)
    pltpu.sync_copy(x_hbm.at[pl.ds(row0, BPW), pl.ds(F1, F2)], idx_b)

    bufs = ((b0a, b0b), (b1a, b1b))
    semsa = (sa0, sa1)
    semsb = (sb0, sb1)
    inv_l = 1.0 / L

    for p in range(2):
        pltpu.async_copy(table_hbm.at[idx_a.at[p]], bufs[p][0], semsa[p])
        pltpu.async_copy(table_hbm.at[idx_b.at[p]], bufs[p][1], semsb[p])

    def body(g, carry):
        for p in range(2):
            b = 2 * g + p
            ba, bb = bufs[p]
            lo = [jnp.zeros((16,), jnp.float32) for _ in range(4)]
            hi = [jnp.zeros((16,), jnp.float32) for _ in range(4)]
            pltpu.make_async_copy(table_hbm.at[idx_a.at[0]], ba,
                                  semsa[p]).wait()
            for j in range(F1):
                c = j % 4
                lo[c] = lo[c] + ba[j, pl.ds(0, 16)]
                hi[c] = hi[c] + ba[j, pl.ds(16, 16)]
            pltpu.make_async_copy(table_hbm.at[idx_b.at[0]], bb,
                                  semsb[p]).wait()
            for j in range(F2):
                c = j % 4
                lo[c] = lo[c] + bb[j, pl.ds(0, 16)]
                hi[c] = hi[c] + bb[j, pl.ds(16, 16)]
            out_v[b, pl.ds(0, 16)] = ((lo[0] + lo[1]) + (lo[2] + lo[3])) * inv_l
            out_v[b, pl.ds(16, 16)] = ((hi[0] + hi[1]) + (hi[2] + hi[3])) * inv_l

            @pl.when(b + 2 < BPW)
            def _():
                pltpu.async_copy(table_hbm.at[idx_a.at[b + 2]], ba, semsa[p])
                pltpu.async_copy(table_hbm.at[idx_b.at[b + 2]], bb, semsb[p])
        return carry

    lax.fori_loop(0, BPW // 2, body, 0)
    pltpu.sync_copy(out_v, avg_hbm.at[pl.ds(wid * BPW, BPW)])


def _linear_body(avg_ref, wt_ref, bias_ref, out_ref):
    out_ref[...] = (
        jnp.dot(avg_ref[...], wt_ref[...], preferred_element_type=jnp.float32)
        + bias_ref[...]
    )


def kernel(x, table, W_out, b_out):
    avg = _pool_kernel(x.astype(jnp.int32), table)
    out = pl.pallas_call(
        _linear_body,
        out_shape=jax.ShapeDtypeStruct((B, NCLS), jnp.float32),
    )(avg, W_out.T, b_out.reshape(1, NCLS))
    return out
